# baseline (speedup 1.0000x reference)
import functools

import numpy as np
import jax
import jax.numpy as jnp
from jax import lax
from jax.experimental import pallas as pl
from jax.experimental.pallas import tpu as pltpu
from jax.experimental.pallas import tpu_sc as plsc

B = 1024
S_IN = 128
T_IN = 256
D = 64
K = 512
NDATA = 1000000
TT = 0.07
EPS = 1e-07
RES = float(K) / float(NDATA)

KP = 560
NCHUNK = 5
CH = 112
NC = 2
NS = 16
NW = NC * NS
RPW = B // NW
NVALID = K + 1


def _butterfly_row_order():
    iota = np.arange(16)
    vs = [np.full(16, float(j)) for j in range(16)]

    def comb(a, b, h):
        sel = ((iota // h) % 2) == 0
        return np.where(sel, a, b) + np.where(sel, b, a)[iota ^ h]

    level = vs
    for h in (8, 4, 2, 1):
        level = [comb(level[2 * i], level[2 * i + 1], h)
                 for i in range(len(level) // 2)]
    pi = (level[0] / 16.0).astype(int)
    A = [0] * 16
    for l in range(16):
        A[pi[l]] = l
    return A


_ROW_OF_ACC = _butterfly_row_order()


def _emb_body(fs_ref, ft_ref, ws_ref, bs_ref, wt_ref, bt_ref, es_ref, et_ref):
    es = jnp.dot(fs_ref[...], ws_ref[...],
                 preferred_element_type=jnp.float32) + bs_ref[...]
    ns = jnp.sqrt(jnp.sum(es * es, axis=1, keepdims=True))
    es_ref[...] = es / (jnp.maximum(ns, 1e-12) * TT)
    et = jnp.dot(ft_ref[...], wt_ref[...],
                 preferred_element_type=jnp.float32) + bt_ref[...]
    nt = jnp.sqrt(jnp.sum(et * et, axis=1, keepdims=True))
    et_ref[...] = et / (jnp.maximum(nt, 1e-12) * TT)


def _embeddings(f_s, f_t, W_s, b_s, W_t, b_t):
    return pl.pallas_call(
        _emb_body,
        out_shape=(jax.ShapeDtypeStruct((B, D), jnp.float32),
                   jax.ShapeDtypeStruct((B, D), jnp.float32)),
    )(f_s, f_t, W_s, b_s.reshape(1, D), W_t, b_t.reshape(1, D))


CBLK = 8000
CHALF = CBLK // 2


def _compact_body(in_ref, o_ref):
    x = in_ref[...]
    o_ref[:, 0:64] = x[0:CHALF, :]
    o_ref[:, 64:128] = x[CHALF:CBLK, :]


def _compact(mem):
    out = pl.pallas_call(
        _compact_body,
        grid=(NDATA // CBLK,),
        in_specs=[pl.BlockSpec((CBLK, D), lambda i: (i, 0))],
        out_specs=pl.BlockSpec((CHALF, 2 * D), lambda i: (i, 0)),
        out_shape=jax.ShapeDtypeStruct((NDATA // 2, 2 * D), jnp.float32),
    )(mem)
    return jnp.reshape(out, (NDATA, D))


def _remap_idx(m):
    return 2 * ((m // CBLK) * CHALF + (m % CHALF)) + ((m % CBLK) // CHALF)


def _combine(a, b, h, iota):
    sel = (iota & h) == 0
    x = iota ^ h
    return (jnp.where(sel, a, b)
            + jnp.where(sel, b, a).at[x].get(mode='promise_in_bounds'))


def _reduce16(accs, iota):
    level = accs
    for h in (8, 4, 2, 1):
        level = [_combine(level[2 * i], level[2 * i + 1], h, iota)
                 for i in range(len(level) // 2)]
    return level[0]


def _sc_gather_dot(memory_v1, memory_v2, idx_pad, emb_t, emb_s):
    mesh = plsc.VectorSubcoreMesh(core_axis_name="c", subcore_axis_name="s",
                                  num_cores=NC, num_subcores=NS)

    scratch = (
        [pltpu.VMEM((RPW * KP,), jnp.int32)]
        + [pltpu.VMEM((RPW, D), jnp.float32)] * 2
        + [pltpu.VMEM((CH, D), jnp.float32)] * NCHUNK
        + [pltpu.VMEM((KP,), jnp.float32)] * 2
        + [pltpu.SemaphoreType.DMA] * NCHUNK
        + [pltpu.SemaphoreType.DMA] * 2
    )

    @functools.partial(
        pl.kernel,
        out_type=(jax.ShapeDtypeStruct((B * KP,), jnp.float32),
                  jax.ShapeDtypeStruct((B * KP,), jnp.float32)),
        mesh=mesh,
        scratch_types=scratch,
        compiler_params=pltpu.CompilerParams(use_tc_tiling_on_sc=False),
    )
    def body(mem1, mem2, idxp, embt, embs, lt_out, ls_out,
             idx_v, embt_v, embs_v, b0, b1, b2, b3, b4, lb0, lb1,
             g0, g1, g2, g3, g4, o0, o1):
        bufs = (b0, b1, b2, b3, b4)
        gsems = (g0, g1, g2, g3, g4)
        wid = lax.axis_index("s") * NC + lax.axis_index("c")
        r0 = wid * RPW
        pltpu.sync_copy(idxp.at[pl.ds(r0 * KP, RPW * KP)], idx_v)
        pltpu.sync_copy(embt.at[pl.ds(r0, RPW), :], embt_v)
        pltpu.sync_copy(embs.at[pl.ds(r0, RPW), :], embs_v)

        def gather_desc(mem, uu, c):
            isl = idx_v.at[pl.ds(uu * KP + c * CH, CH)]
            return pltpu.make_async_copy(mem.at[isl], bufs[c], gsems[c])

        for c in range(NCHUNK):
            gather_desc(mem1, 0, c).start()

        def task(u, bank):
            mem = mem1 if bank == 0 else mem2
            emb_v = embt_v if bank == 0 else embs_v
            out_hbm = lt_out if bank == 0 else ls_out
            lbuf = lb0 if bank == 0 else lb1
            osem = o0 if bank == 0 else o1

            e = [emb_v[u, pl.ds(16 * j, 16)] for j in range(4)]

            @pl.when(u >= 1)
            def _():
                pltpu.make_async_copy(
                    lbuf, out_hbm.at[pl.ds((r0 + u - 1) * KP, KP)],
                    osem).wait()

            for c in range(NCHUNK):
                gather_desc(mem, u, c).wait()
                buf = bufs[c]

                def group(g, _, _c=c, _buf=buf, _lbuf=lbuf, _e=e):
                    iota = lax.iota(jnp.int32, 16)
                    base = g * 16
                    accs = []
                    for j in range(16):
                        row = base + _ROW_OF_ACC[j]
                        acc = _buf[row, pl.ds(0, 16)] * _e[0]
                        acc = acc + _buf[row, pl.ds(16, 16)] * _e[1]
                        acc = acc + _buf[row, pl.ds(32, 16)] * _e[2]
                        acc = acc + _buf[row, pl.ds(48, 16)] * _e[3]
                        accs.append(acc)
                    out16 = _reduce16(accs, iota)
                    _lbuf[pl.ds(_c * CH + base, 16)] = out16
                    return 0

                lax.fori_loop(0, CH // 16, group, 0)

                if bank == 0:
                    gather_desc(mem2, u, c).start()
                else:
                    @pl.when(u + 1 < RPW)
                    def _(_c=c):
                        gather_desc(mem1, u + 1, _c).start()

            pltpu.make_async_copy(
                lbuf, out_hbm.at[pl.ds((r0 + u) * KP, KP)], osem).start()

        def row_body(u, carry):
            task(u, 0)
            task(u, 1)
            return carry

        lax.fori_loop(0, RPW, row_body, 0)

        pltpu.make_async_copy(
            lb0, lt_out.at[pl.ds((r0 + RPW - 1) * KP, KP)], o0).wait()
        pltpu.make_async_copy(
            lb1, ls_out.at[pl.ds((r0 + RPW - 1) * KP, KP)], o1).wait()

    return body(memory_v1, memory_v2, idx_pad, emb_t, emb_s)


LROWS = B * KP // 128


def _loss_body(ls_ref, lt_ref, out_ref):
    flat = (lax.broadcasted_iota(jnp.int32, (LROWS, 128), 0) * 128
            + lax.broadcasted_iota(jnp.int32, (LROWS, 128), 1))
    col = flat % KP
    valid = col < NVALID
    neg = (col >= 1) & valid
    pos = col == 0

    def half(logits):
        ex = jnp.where(valid, jnp.exp(logits), 0.0)
        Z = jnp.sum(ex) * (float(NDATA) / (B * NVALID))
        P = ex / Z
        t1 = jnp.sum(jnp.where(pos, jnp.log(P / (P + RES + EPS)), 0.0))
        t0 = jnp.sum(jnp.where(neg, jnp.log(RES / (P + RES + EPS)), 0.0))
        return -(t1 + t0) / B

    out_ref[...] = jnp.reshape(half(ls_ref[...]) + half(lt_ref[...]), (1, 1))


def _loss(logit_s, logit_t):
    out = pl.pallas_call(
        _loss_body,
        out_shape=jax.ShapeDtypeStruct((1, 1), jnp.float32),
    )(logit_s.reshape(LROWS, 128), logit_t.reshape(LROWS, 128))
    return out[0, 0]


def kernel(f_s, f_t, idx, contrast_idx, memory_v1, memory_v2,
           W_s, b_s, W_t, b_t):
    emb_s, emb_t = _embeddings(f_s, f_t, W_s, b_s, W_t, b_t)
    idx_pad = _remap_idx(jnp.concatenate(
        [idx.astype(jnp.int32)[:, None],
         contrast_idx.astype(jnp.int32),
         contrast_idx[:, :KP - NVALID].astype(jnp.int32)], axis=1))
    logit_t, logit_s = _sc_gather_dot(_compact(memory_v1),
                                      _compact(memory_v2),
                                      idx_pad.reshape(B * KP),
                                      emb_t, emb_s)
    return _loss(logit_s, logit_t)


# speedup vs baseline: 1.0924x; 1.0924x over previous
import functools

import numpy as np
import jax
import jax.numpy as jnp
from jax import lax
from jax.experimental import pallas as pl
from jax.experimental.pallas import tpu as pltpu
from jax.experimental.pallas import tpu_sc as plsc

B = 1024
S_IN = 128
T_IN = 256
D = 64
K = 512
NDATA = 1000000
TT = 0.07
EPS = 1e-07
RES = float(K) / float(NDATA)

KP = 560
NCHUNK = 5
CH = 112
NC = 2
NS = 16
NW = NC * NS
RPW = B // NW
NVALID = K + 1


def _butterfly_row_order():
    iota = np.arange(16)
    vs = [np.full(16, float(j)) for j in range(16)]

    def comb(a, b, h):
        sel = ((iota // h) % 2) == 0
        return np.where(sel, a, b) + np.where(sel, b, a)[iota ^ h]

    level = vs
    for h in (8, 4, 2, 1):
        level = [comb(level[2 * i], level[2 * i + 1], h)
                 for i in range(len(level) // 2)]
    pi = (level[0] / 16.0).astype(int)
    A = [0] * 16
    for l in range(16):
        A[pi[l]] = l
    return A


_ROW_OF_ACC = _butterfly_row_order()


def _emb_body(fs_ref, ft_ref, ws_ref, bs_ref, wt_ref, bt_ref, es_ref, et_ref):
    es = jnp.dot(fs_ref[...], ws_ref[...],
                 preferred_element_type=jnp.float32) + bs_ref[...]
    ns = jnp.sqrt(jnp.sum(es * es, axis=1, keepdims=True))
    es_ref[...] = es / (jnp.maximum(ns, 1e-12) * TT)
    et = jnp.dot(ft_ref[...], wt_ref[...],
                 preferred_element_type=jnp.float32) + bt_ref[...]
    nt = jnp.sqrt(jnp.sum(et * et, axis=1, keepdims=True))
    et_ref[...] = et / (jnp.maximum(nt, 1e-12) * TT)


def _embeddings(f_s, f_t, W_s, b_s, W_t, b_t):
    return pl.pallas_call(
        _emb_body,
        out_shape=(jax.ShapeDtypeStruct((B, D), jnp.float32),
                   jax.ShapeDtypeStruct((B, D), jnp.float32)),
    )(f_s, f_t, W_s, b_s.reshape(1, D), W_t, b_t.reshape(1, D))


def _combine(a, b, h, iota):
    sel = (iota & h) == 0
    x = iota ^ h
    return (jnp.where(sel, a, b)
            + jnp.where(sel, b, a).at[x].get(mode='promise_in_bounds'))


def _reduce16(accs, iota):
    level = accs
    for h in (8, 4, 2, 1):
        level = [_combine(level[2 * i], level[2 * i + 1], h, iota)
                 for i in range(len(level) // 2)]
    return level[0]


def _sc_gather_dot(memory_v1, memory_v2, idx_pad, emb_t, emb_s):
    mesh = plsc.VectorSubcoreMesh(core_axis_name="c", subcore_axis_name="s",
                                  num_cores=NC, num_subcores=NS)

    scratch = (
        [pltpu.VMEM((RPW * KP,), jnp.int32)]
        + [pltpu.VMEM((RPW, D), jnp.float32)] * 2
        + [pltpu.VMEM((CH, D), jnp.float32)] * NCHUNK
        + [pltpu.VMEM((KP,), jnp.float32)] * 2
        + [pltpu.SemaphoreType.DMA] * NCHUNK
        + [pltpu.SemaphoreType.DMA] * 2
    )

    @functools.partial(
        pl.kernel,
        out_type=(jax.ShapeDtypeStruct((B * KP,), jnp.float32),
                  jax.ShapeDtypeStruct((B * KP,), jnp.float32)),
        mesh=mesh,
        scratch_types=scratch,
        compiler_params=pltpu.CompilerParams(use_tc_tiling_on_sc=False),
    )
    def body(mem1, mem2, idxp, embt, embs, lt_out, ls_out,
             idx_v, embt_v, embs_v, b0, b1, b2, b3, b4, lb0, lb1,
             g0, g1, g2, g3, g4, o0, o1):
        bufs = (b0, b1, b2, b3, b4)
        gsems = (g0, g1, g2, g3, g4)
        wid = lax.axis_index("s") * NC + lax.axis_index("c")
        r0 = wid * RPW
        pltpu.sync_copy(idxp.at[pl.ds(r0 * KP, RPW * KP)], idx_v)
        pltpu.sync_copy(embt.at[pl.ds(r0, RPW), :], embt_v)
        pltpu.sync_copy(embs.at[pl.ds(r0, RPW), :], embs_v)

        def gather_desc(mem, uu, c):
            isl = idx_v.at[pl.ds(uu * KP + c * CH, CH)]
            return pltpu.make_async_copy(mem.at[isl], bufs[c], gsems[c])

        for c in range(NCHUNK):
            gather_desc(mem1, 0, c).start()

        def task(u, bank):
            mem = mem1 if bank == 0 else mem2
            emb_v = embt_v if bank == 0 else embs_v
            out_hbm = lt_out if bank == 0 else ls_out
            lbuf = lb0 if bank == 0 else lb1
            osem = o0 if bank == 0 else o1

            e = [emb_v[u, pl.ds(16 * j, 16)] for j in range(4)]

            @pl.when(u >= 1)
            def _():
                pltpu.make_async_copy(
                    lbuf, out_hbm.at[pl.ds((r0 + u - 1) * KP, KP)],
                    osem).wait()

            for c in range(NCHUNK):
                gather_desc(mem, u, c).wait()
                buf = bufs[c]

                def group(g, _, _c=c, _buf=buf, _lbuf=lbuf, _e=e):
                    iota = lax.iota(jnp.int32, 16)
                    base = g * 16
                    accs = []
                    for j in range(16):
                        row = base + _ROW_OF_ACC[j]
                        acc = _buf[row, pl.ds(0, 16)] * _e[0]
                        acc = acc + _buf[row, pl.ds(16, 16)] * _e[1]
                        acc = acc + _buf[row, pl.ds(32, 16)] * _e[2]
                        acc = acc + _buf[row, pl.ds(48, 16)] * _e[3]
                        accs.append(acc)
                    out16 = _reduce16(accs, iota)
                    _lbuf[pl.ds(_c * CH + base, 16)] = out16
                    return 0

                lax.fori_loop(0, CH // 16, group, 0)

                if bank == 0:
                    gather_desc(mem2, u, c).start()
                else:
                    @pl.when(u + 1 < RPW)
                    def _(_c=c):
                        gather_desc(mem1, u + 1, _c).start()

            pltpu.make_async_copy(
                lbuf, out_hbm.at[pl.ds((r0 + u) * KP, KP)], osem).start()

        def row_body(u, carry):
            task(u, 0)
            task(u, 1)
            return carry

        lax.fori_loop(0, RPW, row_body, 0)

        pltpu.make_async_copy(
            lb0, lt_out.at[pl.ds((r0 + RPW - 1) * KP, KP)], o0).wait()
        pltpu.make_async_copy(
            lb1, ls_out.at[pl.ds((r0 + RPW - 1) * KP, KP)], o1).wait()

    return body(memory_v1, memory_v2, idx_pad, emb_t, emb_s)


LROWS = B * KP // 128


def _loss_body(ls_ref, lt_ref, out_ref):
    flat = (lax.broadcasted_iota(jnp.int32, (LROWS, 128), 0) * 128
            + lax.broadcasted_iota(jnp.int32, (LROWS, 128), 1))
    col = flat % KP
    valid = col < NVALID
    neg = (col >= 1) & valid
    pos = col == 0

    def half(logits):
        ex = jnp.where(valid, jnp.exp(logits), 0.0)
        Z = jnp.sum(ex) * (float(NDATA) / (B * NVALID))
        P = ex / Z
        t1 = jnp.sum(jnp.where(pos, jnp.log(P / (P + RES + EPS)), 0.0))
        t0 = jnp.sum(jnp.where(neg, jnp.log(RES / (P + RES + EPS)), 0.0))
        return -(t1 + t0) / B

    out_ref[...] = jnp.reshape(half(ls_ref[...]) + half(lt_ref[...]), (1, 1))


def _loss(logit_s, logit_t):
    out = pl.pallas_call(
        _loss_body,
        out_shape=jax.ShapeDtypeStruct((1, 1), jnp.float32),
    )(logit_s.reshape(LROWS, 128), logit_t.reshape(LROWS, 128))
    return out[0, 0]


def kernel(f_s, f_t, idx, contrast_idx, memory_v1, memory_v2,
           W_s, b_s, W_t, b_t):
    emb_s, emb_t = _embeddings(f_s, f_t, W_s, b_s, W_t, b_t)
    idx_pad = jnp.concatenate(
        [idx.astype(jnp.int32)[:, None],
         contrast_idx.astype(jnp.int32),
         contrast_idx[:, :KP - NVALID].astype(jnp.int32)], axis=1)
    logit_t, logit_s = _sc_gather_dot(memory_v1, memory_v2,
                                      idx_pad.reshape(B * KP),
                                      emb_t, emb_s)
    return _loss(logit_s, logit_t)


# speedup vs baseline: 1.1272x; 1.0319x over previous
import functools

import numpy as np
import jax
import jax.numpy as jnp
from jax import lax
from jax.experimental import pallas as pl
from jax.experimental.pallas import tpu as pltpu
from jax.experimental.pallas import tpu_sc as plsc

B = 1024
S_IN = 128
T_IN = 256
D = 64
K = 512
NDATA = 1000000
TT = 0.07
EPS = 1e-07
RES = float(K) / float(NDATA)

KP = 560
NCHUNK = 5
CH = 112
NC = 2
NS = 16
NW = NC * NS
RPW = B // NW
NVALID = K + 1


def _butterfly_row_order():
    iota = np.arange(16)
    vs = [np.full(16, float(j)) for j in range(16)]

    def comb(a, b, h):
        sel = ((iota // h) % 2) == 0
        return np.where(sel, a, b) + np.where(sel, b, a)[iota ^ h]

    level = vs
    for h in (8, 4, 2, 1):
        level = [comb(level[2 * i], level[2 * i + 1], h)
                 for i in range(len(level) // 2)]
    pi = (level[0] / 16.0).astype(int)
    A = [0] * 16
    for l in range(16):
        A[pi[l]] = l
    return A


_ROW_OF_ACC = _butterfly_row_order()


def _emb_body(fs_ref, ft_ref, ws_ref, bs_ref, wt_ref, bt_ref, es_ref, et_ref):
    es = jnp.dot(fs_ref[...], ws_ref[...],
                 preferred_element_type=jnp.float32) + bs_ref[...]
    ns = jnp.sqrt(jnp.sum(es * es, axis=1, keepdims=True))
    es_ref[...] = es / (jnp.maximum(ns, 1e-12) * TT)
    et = jnp.dot(ft_ref[...], wt_ref[...],
                 preferred_element_type=jnp.float32) + bt_ref[...]
    nt = jnp.sqrt(jnp.sum(et * et, axis=1, keepdims=True))
    et_ref[...] = et / (jnp.maximum(nt, 1e-12) * TT)


def _embeddings(f_s, f_t, W_s, b_s, W_t, b_t):
    return pl.pallas_call(
        _emb_body,
        out_shape=(jax.ShapeDtypeStruct((B, D), jnp.float32),
                   jax.ShapeDtypeStruct((B, D), jnp.float32)),
    )(f_s, f_t, W_s, b_s.reshape(1, D), W_t, b_t.reshape(1, D))


def _combine(a, b, h, iota):
    sel = (iota & h) == 0
    x = iota ^ h
    return (jnp.where(sel, a, b)
            + jnp.where(sel, b, a).at[x].get(mode='promise_in_bounds'))


def _reduce16(accs, iota):
    level = accs
    for h in (8, 4, 2, 1):
        level = [_combine(level[2 * i], level[2 * i + 1], h, iota)
                 for i in range(len(level) // 2)]
    return level[0]


def _sc_gather_dot_one(mem, idx_pad, emb):
    mesh = plsc.VectorSubcoreMesh(core_axis_name="c", subcore_axis_name="s",
                                  num_cores=NC, num_subcores=NS)

    scratch = (
        [pltpu.VMEM((RPW * KP,), jnp.int32)]
        + [pltpu.VMEM((RPW, D), jnp.float32)]
        + [pltpu.VMEM((CH, D), jnp.float32)] * NCHUNK
        + [pltpu.VMEM((KP,), jnp.float32)]
        + [pltpu.SemaphoreType.DMA] * NCHUNK
        + [pltpu.SemaphoreType.DMA]
    )

    @functools.partial(
        pl.kernel,
        out_type=jax.ShapeDtypeStruct((B * KP,), jnp.float32),
        mesh=mesh,
        scratch_types=scratch,
        compiler_params=pltpu.CompilerParams(use_tc_tiling_on_sc=False),
    )
    def body(memr, idxp, embr, out_hbm,
             idx_v, emb_v, b0, b1, b2, b3, b4, lbuf,
             g0, g1, g2, g3, g4, osem):
        bufs = (b0, b1, b2, b3, b4)
        gsems = (g0, g1, g2, g3, g4)
        wid = lax.axis_index("s") * NC + lax.axis_index("c")
        r0 = wid * RPW
        pltpu.sync_copy(idxp.at[pl.ds(r0 * KP, RPW * KP)], idx_v)
        pltpu.sync_copy(embr.at[pl.ds(r0, RPW), :], emb_v)

        def gather_desc(uu, c):
            isl = idx_v.at[pl.ds(uu * KP + c * CH, CH)]
            return pltpu.make_async_copy(memr.at[isl], bufs[c], gsems[c])

        for c in range(NCHUNK):
            gather_desc(0, c).start()

        def row_body(u, carry):
            e = [emb_v[u, pl.ds(16 * j, 16)] for j in range(4)]

            @pl.when(u >= 1)
            def _():
                pltpu.make_async_copy(
                    lbuf, out_hbm.at[pl.ds((r0 + u - 1) * KP, KP)],
                    osem).wait()

            for c in range(NCHUNK):
                gather_desc(u, c).wait()
                buf = bufs[c]

                def group(g, _, _c=c, _buf=buf, _e=e):
                    iota = lax.iota(jnp.int32, 16)
                    base = g * 16
                    accs = []
                    for j in range(16):
                        row = base + _ROW_OF_ACC[j]
                        acc = _buf[row, pl.ds(0, 16)] * _e[0]
                        acc = acc + _buf[row, pl.ds(16, 16)] * _e[1]
                        acc = acc + _buf[row, pl.ds(32, 16)] * _e[2]
                        acc = acc + _buf[row, pl.ds(48, 16)] * _e[3]
                        accs.append(acc)
                    out16 = _reduce16(accs, iota)
                    lbuf[pl.ds(_c * CH + base, 16)] = out16
                    return 0

                lax.fori_loop(0, CH // 16, group, 0)

                @pl.when(u + 1 < RPW)
                def _(_c=c):
                    gather_desc(u + 1, _c).start()

            pltpu.make_async_copy(
                lbuf, out_hbm.at[pl.ds((r0 + u) * KP, KP)], osem).start()
            return carry

        lax.fori_loop(0, RPW, row_body, 0)

        pltpu.make_async_copy(
            lbuf, out_hbm.at[pl.ds((r0 + RPW - 1) * KP, KP)], osem).wait()

    return body(mem, idx_pad, emb)


def _sc_gather_dot(memory_v1, memory_v2, idx_pad, emb_t, emb_s):
    logit_t = _sc_gather_dot_one(memory_v1, idx_pad, emb_t)
    logit_s = _sc_gather_dot_one(memory_v2, idx_pad, emb_s)
    return logit_t, logit_s


LROWS = B * KP // 128


def _loss_body(ls_ref, lt_ref, out_ref):
    flat = (lax.broadcasted_iota(jnp.int32, (LROWS, 128), 0) * 128
            + lax.broadcasted_iota(jnp.int32, (LROWS, 128), 1))
    col = flat % KP
    valid = col < NVALID
    neg = (col >= 1) & valid
    pos = col == 0

    def half(logits):
        ex = jnp.where(valid, jnp.exp(logits), 0.0)
        Z = jnp.sum(ex) * (float(NDATA) / (B * NVALID))
        P = ex / Z
        t1 = jnp.sum(jnp.where(pos, jnp.log(P / (P + RES + EPS)), 0.0))
        t0 = jnp.sum(jnp.where(neg, jnp.log(RES / (P + RES + EPS)), 0.0))
        return -(t1 + t0) / B

    out_ref[...] = jnp.reshape(half(ls_ref[...]) + half(lt_ref[...]), (1, 1))


def _loss(logit_s, logit_t):
    out = pl.pallas_call(
        _loss_body,
        out_shape=jax.ShapeDtypeStruct((1, 1), jnp.float32),
    )(logit_s.reshape(LROWS, 128), logit_t.reshape(LROWS, 128))
    return out[0, 0]


def kernel(f_s, f_t, idx, contrast_idx, memory_v1, memory_v2,
           W_s, b_s, W_t, b_t):
    emb_s, emb_t = _embeddings(f_s, f_t, W_s, b_s, W_t, b_t)
    idx_pad = jnp.concatenate(
        [idx.astype(jnp.int32)[:, None],
         contrast_idx.astype(jnp.int32),
         contrast_idx[:, :KP - NVALID].astype(jnp.int32)], axis=1)
    logit_t, logit_s = _sc_gather_dot(memory_v1, memory_v2,
                                      idx_pad.reshape(B * KP),
                                      emb_t, emb_s)
    return _loss(logit_s, logit_t)
